# Initial kernel scaffold; baseline (speedup 1.0000x reference)
import jax
import jax.numpy as jnp
from jax.experimental import pallas as pl


def kernel(x, edge_index, edge_label_index, W1, b1, W2, b2):
    raise NotImplementedError("write your pallas kernel here")


# speedup vs baseline: 9.2614x; 9.2614x over previous
import functools

import jax
import jax.numpy as jnp
from jax import lax
from jax.experimental import pallas as pl
from jax.experimental.pallas import tpu as pltpu
from jax.experimental.pallas import tpu_sc as plsc

N_NODES = 10000
N_EDGES = 320000
N_LABEL = 100000
IN_CH = 128
HID_CH = 64

NC = 2
NS = 16
IDXW = 128

N_PAD = 10112
E_SUB = 80
E_PAD = NC * NS * E_SUB * IDXW
L_SUB = 25
L_PAD = NC * NS * L_SUB * IDXW

_mesh = plsc.VectorSubcoreMesh(core_axis_name="c", subcore_axis_name="s")
_sc_params = pltpu.CompilerParams(use_tc_tiling_on_sc=False)


def _sc_deg(dst2d, zeros16, ones16):
    rows_pt = N_PAD // NS

    @functools.partial(
        pl.kernel,
        out_type=[jax.ShapeDtypeStruct((N_PAD, 16), jnp.float32)] * 2,
        mesh=_mesh,
        compiler_params=_sc_params,
        scratch_types=[
            pltpu.VMEM((E_SUB, IDXW), jnp.int32),
            pltpu.VMEM((IDXW, 16), jnp.float32),
            pltpu.VMEM_SHARED((N_PAD, 16), jnp.float32),
        ],
    )
    def k(dst_hbm, z_hbm, ones_hbm, out_a, out_b, idx_v, ones_v, acc_sh):
        c = lax.axis_index("c")
        s = lax.axis_index("s")
        sl = pl.ds(s * rows_pt, rows_pt)
        pltpu.sync_copy(z_hbm.at[sl], acc_sh.at[sl])
        pltpu.sync_copy(ones_hbm, ones_v)
        base = (c * NS + s) * E_SUB
        pltpu.sync_copy(dst_hbm.at[pl.ds(base, E_SUB)], idx_v)
        plsc.subcore_barrier()

        @pl.loop(0, E_SUB)
        def _(j):
            pltpu.sync_copy(ones_v, acc_sh.at[idx_v.at[j]], add=True)

        plsc.subcore_barrier()

        @pl.when(c == 0)
        def _():
            pltpu.sync_copy(acc_sh.at[sl], out_a.at[sl])

        @pl.when(c == 1)
        def _():
            pltpu.sync_copy(acc_sh.at[sl], out_b.at[sl])

    return k(dst2d, zeros16, ones16)


def _sc_agg(g_pad, src2d, dst2d, zeros64):
    rows_pt = N_PAD // NS

    @functools.partial(
        pl.kernel,
        out_type=[jax.ShapeDtypeStruct((N_PAD, HID_CH), jnp.float32)] * 2,
        mesh=_mesh,
        compiler_params=_sc_params,
        scratch_types=[
            pltpu.VMEM((E_SUB, IDXW), jnp.int32),
            pltpu.VMEM((E_SUB, IDXW), jnp.int32),
            pltpu.VMEM((IDXW, HID_CH), jnp.float32),
            pltpu.VMEM_SHARED((N_PAD, HID_CH), jnp.float32),
        ],
    )
    def k(g_hbm, src_hbm, dst_hbm, z_hbm, out_a, out_b,
          sidx_v, didx_v, rows_v, acc_sh):
        c = lax.axis_index("c")
        s = lax.axis_index("s")
        sl = pl.ds(s * rows_pt, rows_pt)
        pltpu.sync_copy(z_hbm.at[sl], acc_sh.at[sl])
        base = (c * NS + s) * E_SUB
        pltpu.sync_copy(src_hbm.at[pl.ds(base, E_SUB)], sidx_v)
        pltpu.sync_copy(dst_hbm.at[pl.ds(base, E_SUB)], didx_v)
        plsc.subcore_barrier()

        @pl.loop(0, E_SUB)
        def _(j):
            pltpu.sync_copy(g_hbm.at[sidx_v.at[j]], rows_v)
            pltpu.sync_copy(rows_v, acc_sh.at[didx_v.at[j]], add=True)

        plsc.subcore_barrier()

        @pl.when(c == 0)
        def _():
            pltpu.sync_copy(acc_sh.at[sl], out_a.at[sl])

        @pl.when(c == 1)
        def _():
            pltpu.sync_copy(acc_sh.at[sl], out_b.at[sl])

    return k(g_pad, src2d, dst2d, zeros64)


def _sc_score_gather(hf_pad, ia2d, ib2d):
    @functools.partial(
        pl.kernel,
        out_type=[jax.ShapeDtypeStruct((L_PAD, HID_CH), jnp.float32)] * 2,
        mesh=_mesh,
        compiler_params=_sc_params,
        scratch_types=[
            pltpu.VMEM((L_SUB * IDXW,), jnp.int32),
            pltpu.VMEM((L_SUB * IDXW,), jnp.int32),
            pltpu.VMEM((IDXW, HID_CH), jnp.float32),
        ],
    )
    def k(hf_hbm, ia_hbm, ib_hbm, out_a, out_b, ia_v, ib_v, rows_v):
        c = lax.axis_index("c")
        s = lax.axis_index("s")
        wid = c * NS + s
        base = wid * L_SUB * IDXW
        pltpu.sync_copy(ia_hbm.at[pl.ds(base, L_SUB * IDXW)], ia_v)
        pltpu.sync_copy(ib_hbm.at[pl.ds(base, L_SUB * IDXW)], ib_v)

        @pl.loop(0, L_SUB)
        def _(j):
            pltpu.sync_copy(hf_hbm.at[ia_v.at[pl.ds(j * IDXW, IDXW)]], rows_v)
            pltpu.sync_copy(rows_v, out_a.at[pl.ds(base + j * IDXW, IDXW)])
            pltpu.sync_copy(hf_hbm.at[ib_v.at[pl.ds(j * IDXW, IDXW)]], rows_v)
            pltpu.sync_copy(rows_v, out_b.at[pl.ds(base + j * IDXW, IDXW)])

    return k(hf_pad, ia2d, ib2d)


_TC_BLK = 1000


def _dis_block(da_ref, db_ref):
    deg = da_ref[:, 0:1] + db_ref[:, 0:1] + 1.0
    return lax.rsqrt(deg)


def _tc_g1(x, W1, degA, degB):
    def body(x_ref, w_ref, da_ref, db_ref, o_ref):
        h = jnp.dot(x_ref[...], w_ref[...], preferred_element_type=jnp.float32)
        o_ref[...] = _dis_block(da_ref, db_ref) * h

    return pl.pallas_call(
        body,
        grid=(N_NODES // _TC_BLK,),
        in_specs=[
            pl.BlockSpec((_TC_BLK, IN_CH), lambda i: (i, 0)),
            pl.BlockSpec((IN_CH, HID_CH), lambda i: (0, 0)),
            pl.BlockSpec((_TC_BLK, 16), lambda i: (i, 0)),
            pl.BlockSpec((_TC_BLK, 16), lambda i: (i, 0)),
        ],
        out_specs=pl.BlockSpec((_TC_BLK, HID_CH), lambda i: (i, 0)),
        out_shape=jax.ShapeDtypeStruct((N_NODES, HID_CH), jnp.float32),
    )(x, W1, degA, degB)


def _tc_layer2(g1, aggA, aggB, degA, degB, b1, W2):
    def body(g_ref, aa_ref, ab_ref, da_ref, db_ref, b_ref, w_ref, o_ref):
        dis = _dis_block(da_ref, db_ref)
        x1 = dis * (aa_ref[...] + ab_ref[...] + g_ref[...]) + b_ref[...]
        x1 = jnp.maximum(x1, 0.0)
        h2 = jnp.dot(x1, w_ref[...], preferred_element_type=jnp.float32)
        o_ref[...] = dis * h2

    return pl.pallas_call(
        body,
        grid=(N_NODES // _TC_BLK,),
        in_specs=[
            pl.BlockSpec((_TC_BLK, HID_CH), lambda i: (i, 0)),
            pl.BlockSpec((_TC_BLK, HID_CH), lambda i: (i, 0)),
            pl.BlockSpec((_TC_BLK, HID_CH), lambda i: (i, 0)),
            pl.BlockSpec((_TC_BLK, 16), lambda i: (i, 0)),
            pl.BlockSpec((_TC_BLK, 16), lambda i: (i, 0)),
            pl.BlockSpec((1, HID_CH), lambda i: (0, 0)),
            pl.BlockSpec((HID_CH, HID_CH), lambda i: (0, 0)),
        ],
        out_specs=pl.BlockSpec((_TC_BLK, HID_CH), lambda i: (i, 0)),
        out_shape=jax.ShapeDtypeStruct((N_NODES, HID_CH), jnp.float32),
    )(g1, aggA, aggB, degA, degB, b1, W2)


def _tc_final(g2, aggA, aggB, degA, degB, b2):
    def body(g_ref, aa_ref, ab_ref, da_ref, db_ref, b_ref, o_ref):
        dis = _dis_block(da_ref, db_ref)
        o_ref[...] = dis * (aa_ref[...] + ab_ref[...] + g_ref[...]) + b_ref[...]

    return pl.pallas_call(
        body,
        grid=(N_NODES // _TC_BLK,),
        in_specs=[
            pl.BlockSpec((_TC_BLK, HID_CH), lambda i: (i, 0)),
            pl.BlockSpec((_TC_BLK, HID_CH), lambda i: (i, 0)),
            pl.BlockSpec((_TC_BLK, HID_CH), lambda i: (i, 0)),
            pl.BlockSpec((_TC_BLK, 16), lambda i: (i, 0)),
            pl.BlockSpec((_TC_BLK, 16), lambda i: (i, 0)),
            pl.BlockSpec((1, HID_CH), lambda i: (0, 0)),
        ],
        out_specs=pl.BlockSpec((_TC_BLK, HID_CH), lambda i: (i, 0)),
        out_shape=jax.ShapeDtypeStruct((N_NODES, HID_CH), jnp.float32),
    )(g2, aggA, aggB, degA, degB, b2)


def _tc_score(ha, hb):
    blk = L_PAD // 10

    def body(a_ref, b_ref, o_ref):
        o_ref[...] = jnp.sum(a_ref[...] * b_ref[...], axis=1)

    return pl.pallas_call(
        body,
        grid=(10,),
        in_specs=[
            pl.BlockSpec((blk, HID_CH), lambda i: (i, 0)),
            pl.BlockSpec((blk, HID_CH), lambda i: (i, 0)),
        ],
        out_specs=pl.BlockSpec((blk,), lambda i: (i,)),
        out_shape=jax.ShapeDtypeStruct((L_PAD,), jnp.float32),
    )(ha, hb)


def kernel(x, edge_index, edge_label_index, W1, b1, W2, b2):
    i32 = jnp.int32
    src = edge_index[0].astype(i32)
    dst = edge_index[1].astype(i32)
    e_fill = jnp.full((E_PAD - N_EDGES,), N_NODES, dtype=i32)
    src2d = jnp.concatenate([src, e_fill]).reshape(E_PAD // IDXW, IDXW)
    dst2d = jnp.concatenate([dst, e_fill]).reshape(E_PAD // IDXW, IDXW)

    l_fill = jnp.zeros((L_PAD - N_LABEL,), dtype=i32)
    ia1d = jnp.concatenate([edge_label_index[0].astype(i32), l_fill])
    ib1d = jnp.concatenate([edge_label_index[1].astype(i32), l_fill])

    zeros16 = jnp.zeros((N_PAD, 16), jnp.float32)
    ones16 = jnp.ones((IDXW, 16), jnp.float32)
    zeros64 = jnp.zeros((N_PAD, HID_CH), jnp.float32)
    row_pad = jnp.zeros((N_PAD - N_NODES, HID_CH), jnp.float32)

    degA, degB = _sc_deg(dst2d, zeros16, ones16)
    degA, degB = degA[:N_NODES], degB[:N_NODES]

    g1 = _tc_g1(x, W1, degA, degB)
    g1_pad = jnp.concatenate([g1, row_pad])
    aggA1, aggB1 = _sc_agg(g1_pad, src2d, dst2d, zeros64)

    g2 = _tc_layer2(g1, aggA1[:N_NODES], aggB1[:N_NODES], degA, degB,
                    b1.reshape(1, HID_CH), W2)
    g2_pad = jnp.concatenate([g2, row_pad])
    aggA2, aggB2 = _sc_agg(g2_pad, src2d, dst2d, zeros64)

    hf = _tc_final(g2, aggA2[:N_NODES], aggB2[:N_NODES], degA, degB,
                   b2.reshape(1, HID_CH))
    hf_pad = jnp.concatenate([hf, row_pad])

    ha, hb = _sc_score_gather(hf_pad, ia1d, ib1d)
    out = _tc_score(ha, hb)
    return out[:N_LABEL]
